# baseline (speedup 1.0000x reference)
import jax
import jax.numpy as jnp
from jax.experimental import pallas as pl
from jax.experimental.pallas import tpu as pltpu

BM = 1024
NBUF = 4


def _body(x_hbm, w1_ref, b1_ref, w2_ref, b2_ref, w3_ref, b3_ref,
          w4_ref, b4_ref, o_hbm):
    n_tiles = x_hbm.shape[0] // BM

    def inner(x_ref, o_ref):
        x = x_ref[...].astype(jnp.bfloat16)
        h = jnp.dot(x, w1_ref[...],
                    preferred_element_type=jnp.float32) + b1_ref[...]
        h = jnp.maximum(h, 0.0)
        h = jnp.dot(h.astype(jnp.bfloat16), w2_ref[...],
                    preferred_element_type=jnp.float32) + b2_ref[...]
        h = h * jax.nn.sigmoid(h)
        h = jnp.dot(h.astype(jnp.bfloat16), w3_ref[...],
                    preferred_element_type=jnp.float32) + b3_ref[...]
        h = h * jax.nn.sigmoid(h)
        h = jnp.dot(h.astype(jnp.bfloat16), w4_ref[...],
                    preferred_element_type=jnp.float32) + b4_ref[...]
        o_ref[...] = h

    deep_in = pl.BlockSpec((BM, x_hbm.shape[1]), lambda i: (i, 0),
                           pipeline_mode=pl.Buffered(buffer_count=NBUF))
    deep_out = pl.BlockSpec((BM, o_hbm.shape[1]), lambda i: (i, 0),
                            pipeline_mode=pl.Buffered(buffer_count=2))
    pltpu.emit_pipeline(
        inner,
        grid=(n_tiles,),
        in_specs=[deep_in],
        out_specs=[deep_out],
    )(x_hbm, o_hbm)


def kernel(t, x_flat, W1, b1, W2, b2, W3, b3, W4, b4):
    del t
    B, D = x_flat.shape
    H = W1.shape[1]

    vm = pl.BlockSpec(memory_space=pltpu.MemorySpace.VMEM)
    anym = pl.BlockSpec(memory_space=pltpu.MemorySpace.HBM)

    return pl.pallas_call(
        _body,
        in_specs=[anym, vm, vm, vm, vm, vm, vm, vm, vm],
        out_specs=anym,
        out_shape=jax.ShapeDtypeStruct((B, D), jnp.float32),
    )(x_flat,
      W1.astype(jnp.bfloat16), b1.reshape(1, H),
      W2.astype(jnp.bfloat16), b2.reshape(1, H),
      W3.astype(jnp.bfloat16), b3.reshape(1, H),
      W4.astype(jnp.bfloat16), b4.reshape(1, D))


# speedup vs baseline: 1.1798x; 1.1798x over previous
import jax
import jax.numpy as jnp
from jax.experimental import pallas as pl


def _mlp_body(x_ref, w1_ref, b1_ref, w2_ref, b2_ref, w3_ref, b3_ref,
              w4_ref, b4_ref, o_ref):
    h = jnp.dot(x_ref[...], w1_ref[...],
                preferred_element_type=jnp.float32) + b1_ref[...]
    h = jnp.maximum(h, 0.0)
    h = jnp.dot(h, w2_ref[...], preferred_element_type=jnp.float32) + b2_ref[...]
    h = h * jax.nn.sigmoid(h)
    h = jnp.dot(h, w3_ref[...], preferred_element_type=jnp.float32) + b3_ref[...]
    h = h * jax.nn.sigmoid(h)
    h = jnp.dot(h, w4_ref[...], preferred_element_type=jnp.float32) + b4_ref[...]
    o_ref[...] = h


def kernel(t, x_flat, W1, b1, W2, b2, W3, b3, W4, b4):
    del t
    B, D = x_flat.shape
    H = W1.shape[1]
    BM = 4096
    grid = (B // BM,)

    def full(shape):
        return pl.BlockSpec(shape, lambda i: (0, 0))

    return pl.pallas_call(
        _mlp_body,
        grid=grid,
        in_specs=[
            pl.BlockSpec((BM, D), lambda i: (i, 0)),
            full((D, H)), full((1, H)),
            full((H, H)), full((1, H)),
            full((H, H)), full((1, H)),
            full((H, D)), full((1, D)),
        ],
        out_specs=pl.BlockSpec((BM, D), lambda i: (i, 0)),
        out_shape=jax.ShapeDtypeStruct((B, D), jnp.float32),
    )(x_flat, W1, b1.reshape(1, H), W2, b2.reshape(1, H),
      W3, b3.reshape(1, H), W4, b4.reshape(1, D))
